# Initial kernel scaffold; baseline (speedup 1.0000x reference)
import functools

import jax
import jax.numpy as jnp
from jax import lax
from jax.experimental import pallas as pl
from jax.experimental.pallas import tpu as pltpu
from jax.experimental.pallas import tpu_sc as plsc

N = 10000
NPAD = 10240
E = 320000
LPAD = 204800
D_EMB = 128
D_HID = 128
D_OUT = 64

NC = 2
NS = 16
NW = NC * NS
ROWS_PER_TILE = NPAD // NS

EPW = E // NW
EB = 80
EBATCHES = EPW // EB

LPW = LPAD // NW
LB = 128
LBATCHES = LPW // LB

_MESH = plsc.VectorSubcoreMesh(
    core_axis_name="c", subcore_axis_name="s", num_cores=NC, num_subcores=NS)


def _wid():
    return lax.axis_index("s") * NC + lax.axis_index("c")


def _deg_body(dst_hbm, ones_hbm, zeros_hbm, out_hbm, table, onesv, idxv):
    cid = lax.axis_index("c")
    sid = lax.axis_index("s")
    wid = _wid()
    r0 = sid * ROWS_PER_TILE
    pltpu.sync_copy(zeros_hbm.at[pl.ds(r0, ROWS_PER_TILE)],
                    table.at[pl.ds(r0, ROWS_PER_TILE)])
    pltpu.sync_copy(ones_hbm, onesv)
    plsc.subcore_barrier()

    def body(b, _):
        base = pl.multiple_of(wid * EPW + b * EB, 8)
        pltpu.sync_copy(dst_hbm.at[pl.ds(base, EB)], idxv)
        pltpu.sync_copy(onesv, table.at[idxv], add=True)
        return 0

    lax.fori_loop(0, EBATCHES, body, 0)
    plsc.subcore_barrier()
    pltpu.sync_copy(table.at[pl.ds(r0, ROWS_PER_TILE)],
                    out_hbm.at[cid, pl.ds(r0, ROWS_PER_TILE)])


def _degrees(dst):
    ones = jnp.ones((EB, 16), jnp.float32)
    zeros = jnp.zeros((NPAD, 16), jnp.float32)
    f = pl.kernel(
        _deg_body,
        out_type=jax.ShapeDtypeStruct((NC, NPAD, 16), jnp.float32),
        mesh=_MESH,
        scratch_types=[
            pltpu.VMEM_SHARED((NPAD, 16), jnp.float32),
            pltpu.VMEM((EB, 16), jnp.float32),
            pltpu.VMEM((EB,), jnp.int32),
        ],
    )
    return f(dst, ones, zeros)


def _agg_body(hs_hbm, src_hbm, dst_hbm, zeros_hbm, out_hbm,
              table, idxs, idxd, rows, sem):
    cid = lax.axis_index("c")
    sid = lax.axis_index("s")
    wid = _wid()
    r0 = sid * ROWS_PER_TILE
    pltpu.sync_copy(zeros_hbm.at[pl.ds(r0, ROWS_PER_TILE)],
                    table.at[pl.ds(r0, ROWS_PER_TILE)])
    plsc.subcore_barrier()

    def body(b, _):
        base = pl.multiple_of(wid * EPW + b * EB, 8)
        pltpu.sync_copy(src_hbm.at[pl.ds(base, EB)], idxs)
        pltpu.sync_copy(dst_hbm.at[pl.ds(base, EB)], idxd)
        pltpu.async_copy(hs_hbm.at[idxs], rows, sem).wait()
        pltpu.sync_copy(rows, table.at[idxd], add=True)
        return 0

    lax.fori_loop(0, EBATCHES, body, 0)
    plsc.subcore_barrier()
    pltpu.sync_copy(table.at[pl.ds(r0, ROWS_PER_TILE)],
                    out_hbm.at[cid, pl.ds(r0, ROWS_PER_TILE)])


def _aggregate(hs, src, dst, d):
    zeros = jnp.zeros((NPAD, d), jnp.float32)
    f = pl.kernel(
        _agg_body,
        out_type=jax.ShapeDtypeStruct((NC, NPAD, d), jnp.float32),
        mesh=_MESH,
        scratch_types=[
            pltpu.VMEM_SHARED((NPAD, d), jnp.float32),
            pltpu.VMEM((EB,), jnp.int32),
            pltpu.VMEM((EB,), jnp.int32),
            pltpu.VMEM((EB, d), jnp.float32),
            pltpu.SemaphoreType.DMA,
        ],
    )
    return f(hs, src, dst, zeros)


def _decode_body(z_hbm, sl_hbm, dl_hbm, out_hbm,
                 sidx, didx, srows, drows, outv, sem):
    wid = _wid()
    iota = lax.iota(jnp.int32, 16)

    def body(b, _):
        base = pl.multiple_of(wid * LPW + b * LB, 8)
        pltpu.sync_copy(sl_hbm.at[pl.ds(base, LB)], sidx)
        pltpu.sync_copy(dl_hbm.at[pl.ds(base, LB)], didx)
        cp1 = pltpu.async_copy(z_hbm.at[sidx], srows, sem)
        cp2 = pltpu.async_copy(z_hbm.at[didx], drows, sem)
        cp1.wait()
        cp2.wait()

        def group(g, _):
            row = g * 16 + iota
            acc = jnp.zeros((16,), jnp.float32)
            for c in range(D_OUT):
                col = jnp.full((16,), c, jnp.int32)
                acc = acc + (plsc.load_gather(srows, [row, col]) *
                             plsc.load_gather(drows, [row, col]))
            outv[pl.ds(g * 16, 16)] = acc
            return 0

        lax.fori_loop(0, LB // 16, group, 0)
        pltpu.sync_copy(outv, out_hbm.at[pl.ds(base, LB)])
        return 0

    lax.fori_loop(0, LBATCHES, body, 0)


def _decode(z, sl, dl):
    f = pl.kernel(
        _decode_body,
        out_type=jax.ShapeDtypeStruct((LPAD,), jnp.float32),
        mesh=_MESH,
        scratch_types=[
            pltpu.VMEM((LB,), jnp.int32),
            pltpu.VMEM((LB,), jnp.int32),
            pltpu.VMEM((LB, D_OUT), jnp.float32),
            pltpu.VMEM((LB, D_OUT), jnp.float32),
            pltpu.VMEM((LB,), jnp.float32),
            pltpu.SemaphoreType.DMA,
        ],
    )
    return f(z, sl, dl)


_BLK = 1000


def _t1_body(d0_ref, d1_ref, emb_ref, w1_ref, hs_ref, dinv_ref):
    deg = d0_ref[:, 0:1] + d1_ref[:, 0:1] + 1.0
    dinv = lax.rsqrt(deg)
    h = jnp.dot(emb_ref[...], w1_ref[...], preferred_element_type=jnp.float32)
    hs_ref[...] = h * dinv
    dinv_ref[...] = jnp.broadcast_to(dinv, (_BLK, 16))


def _t1(d0, d1, emb, w1):
    return pl.pallas_call(
        _t1_body,
        grid=(N // _BLK,),
        in_specs=[
            pl.BlockSpec((_BLK, 16), lambda i: (i, 0)),
            pl.BlockSpec((_BLK, 16), lambda i: (i, 0)),
            pl.BlockSpec((_BLK, D_EMB), lambda i: (i, 0)),
            pl.BlockSpec((D_EMB, D_HID), lambda i: (0, 0)),
        ],
        out_specs=[
            pl.BlockSpec((_BLK, D_HID), lambda i: (i, 0)),
            pl.BlockSpec((_BLK, 16), lambda i: (i, 0)),
        ],
        out_shape=[
            jax.ShapeDtypeStruct((N, D_HID), jnp.float32),
            jax.ShapeDtypeStruct((N, 16), jnp.float32),
        ],
    )(d0, d1, emb, w1)


def _t2_body(p0_ref, p1_ref, hs1_ref, dinv_ref, b1_ref, w2_ref, hs2_ref):
    dinv = dinv_ref[:, 0:1]
    agg = (p0_ref[...] + p1_ref[...] + hs1_ref[...]) * dinv
    x2 = jnp.maximum(agg + b1_ref[...], 0.0)
    h2 = jnp.dot(x2, w2_ref[...], preferred_element_type=jnp.float32)
    hs2_ref[...] = h2 * dinv


def _t2(p0, p1, hs1, dinv, b1, w2):
    return pl.pallas_call(
        _t2_body,
        grid=(N // _BLK,),
        in_specs=[
            pl.BlockSpec((_BLK, D_HID), lambda i: (i, 0)),
            pl.BlockSpec((_BLK, D_HID), lambda i: (i, 0)),
            pl.BlockSpec((_BLK, D_HID), lambda i: (i, 0)),
            pl.BlockSpec((_BLK, 16), lambda i: (i, 0)),
            pl.BlockSpec((1, D_HID), lambda i: (0, 0)),
            pl.BlockSpec((D_HID, D_OUT), lambda i: (0, 0)),
        ],
        out_specs=pl.BlockSpec((_BLK, D_OUT), lambda i: (i, 0)),
        out_shape=jax.ShapeDtypeStruct((N, D_OUT), jnp.float32),
    )(p0, p1, hs1, dinv, b1, w2)


def _t3_body(q0_ref, q1_ref, hs2_ref, dinv_ref, b2_ref, z_ref):
    dinv = dinv_ref[:, 0:1]
    z_ref[...] = (q0_ref[...] + q1_ref[...] + hs2_ref[...]) * dinv + b2_ref[...]


def _t3(q0, q1, hs2, dinv, b2):
    return pl.pallas_call(
        _t3_body,
        grid=(N // _BLK,),
        in_specs=[
            pl.BlockSpec((_BLK, D_OUT), lambda i: (i, 0)),
            pl.BlockSpec((_BLK, D_OUT), lambda i: (i, 0)),
            pl.BlockSpec((_BLK, D_OUT), lambda i: (i, 0)),
            pl.BlockSpec((_BLK, 16), lambda i: (i, 0)),
            pl.BlockSpec((1, D_OUT), lambda i: (0, 0)),
        ],
        out_specs=pl.BlockSpec((_BLK, D_OUT), lambda i: (i, 0)),
        out_shape=jax.ShapeDtypeStruct((N, D_OUT), jnp.float32),
    )(q0, q1, hs2, dinv, b2)


def kernel(edge_index, edge_label_index, embedding, W1, b1, W2, b2):
    src = edge_index[0].astype(jnp.int32)
    dst = edge_index[1].astype(jnp.int32)
    npad_l = LPAD - edge_label_index.shape[1]
    sl = jnp.concatenate(
        [edge_label_index[0].astype(jnp.int32), jnp.zeros((npad_l,), jnp.int32)])
    dl = jnp.concatenate(
        [edge_label_index[1].astype(jnp.int32), jnp.zeros((npad_l,), jnp.int32)])

    deg = _degrees(dst)
    hs1, dinv = _t1(deg[0, :N], deg[1, :N], embedding, W1)
    p = _aggregate(hs1, src, dst, D_HID)
    hs2 = _t2(p[0, :N], p[1, :N], hs1, dinv, b1[None, :], W2)
    q = _aggregate(hs2, src, dst, D_OUT)
    z = _t3(q[0, :N], q[1, :N], hs2, dinv, b2[None, :])
    scores = _decode(z, sl, dl)
    return scores[:edge_label_index.shape[1]]


# speedup vs baseline: 7.3812x; 7.3812x over previous
import functools

import jax
import jax.numpy as jnp
from jax import lax
from jax.experimental import pallas as pl
from jax.experimental.pallas import tpu as pltpu
from jax.experimental.pallas import tpu_sc as plsc

N = 10000
NPAD = 10240
E = 320000
LPAD = 204800
D_EMB = 128
D_HID = 128
D_OUT = 64

NC = 2
NS = 16
NW = NC * NS
ROWS_PER_TILE = NPAD // NS

EPW = E // NW
EB = 80
EBATCHES = EPW // EB

LPW = LPAD // NW
LB = 128
LBATCHES = LPW // LB

_MESH = plsc.VectorSubcoreMesh(
    core_axis_name="c", subcore_axis_name="s", num_cores=NC, num_subcores=NS)
_SC_PARAMS = pltpu.CompilerParams(use_tc_tiling_on_sc=False)
_SC_PARAMS_NL = pltpu.CompilerParams(
    use_tc_tiling_on_sc=False, needs_layout_passes=False)


def _wid():
    return lax.axis_index("s") * NC + lax.axis_index("c")


def _deg_body(dst_hbm, ones_hbm, zeros_hbm, out_hbm, table, onesv, idxv):
    cid = lax.axis_index("c")
    sid = lax.axis_index("s")
    wid = _wid()
    r0 = sid * ROWS_PER_TILE
    pltpu.sync_copy(zeros_hbm.at[pl.ds(r0, ROWS_PER_TILE)],
                    table.at[pl.ds(r0, ROWS_PER_TILE)])
    pltpu.sync_copy(ones_hbm, onesv)
    plsc.subcore_barrier()

    def body(b, _):
        base = pl.multiple_of(wid * EPW + b * EB, 8)
        pltpu.sync_copy(dst_hbm.at[pl.ds(base, EB)], idxv)
        pltpu.sync_copy(onesv, table.at[idxv], add=True)
        return 0

    lax.fori_loop(0, EBATCHES, body, 0)
    plsc.subcore_barrier()
    pltpu.sync_copy(table.at[pl.ds(r0, ROWS_PER_TILE)],
                    out_hbm.at[cid, pl.ds(r0, ROWS_PER_TILE)])


def _degrees(dst):
    ones = jnp.ones((EB, 16), jnp.float32)
    zeros = jnp.zeros((NPAD, 16), jnp.float32)
    f = pl.kernel(
        _deg_body,
        out_type=jax.ShapeDtypeStruct((NC, NPAD, 16), jnp.float32),
        mesh=_MESH,
        compiler_params=_SC_PARAMS,
        scratch_types=[
            pltpu.VMEM_SHARED((NPAD, 16), jnp.float32),
            pltpu.VMEM((EB, 16), jnp.float32),
            pltpu.VMEM((EB,), jnp.int32),
        ],
    )
    return f(dst, ones, zeros)


def _agg_body(hs_hbm, src_hbm, dst_hbm, zeros_hbm, out_hbm,
              table, idxs, idxd, rows, sem):
    cid = lax.axis_index("c")
    sid = lax.axis_index("s")
    wid = _wid()
    r0 = sid * ROWS_PER_TILE
    pltpu.sync_copy(zeros_hbm.at[pl.ds(r0, ROWS_PER_TILE)],
                    table.at[pl.ds(r0, ROWS_PER_TILE)])
    plsc.subcore_barrier()

    def body(b, _):
        base = pl.multiple_of(wid * EPW + b * EB, 8)
        pltpu.sync_copy(src_hbm.at[pl.ds(base, EB)], idxs)
        pltpu.sync_copy(dst_hbm.at[pl.ds(base, EB)], idxd)
        pltpu.async_copy(hs_hbm.at[idxs], rows, sem).wait()
        pltpu.sync_copy(rows, table.at[idxd], add=True)
        return 0

    lax.fori_loop(0, EBATCHES, body, 0)
    plsc.subcore_barrier()
    pltpu.sync_copy(table.at[pl.ds(r0, ROWS_PER_TILE)],
                    out_hbm.at[cid, pl.ds(r0, ROWS_PER_TILE)])


def _aggregate(hs, src, dst, d):
    zeros = jnp.zeros((NPAD, d), jnp.float32)
    f = pl.kernel(
        _agg_body,
        out_type=jax.ShapeDtypeStruct((NC, NPAD, d), jnp.float32),
        mesh=_MESH,
        compiler_params=_SC_PARAMS,
        scratch_types=[
            pltpu.VMEM_SHARED((NPAD, d), jnp.float32),
            pltpu.VMEM((EB,), jnp.int32),
            pltpu.VMEM((EB,), jnp.int32),
            pltpu.VMEM((EB, d), jnp.float32),
            pltpu.SemaphoreType.DMA,
        ],
    )
    return f(hs, src, dst, zeros)


def _decode_body(z_hbm, sl_hbm, dl_hbm, out_hbm,
                 sidx, didx, srows, drows, outv, sem):
    wid = _wid()
    iota = lax.iota(jnp.int32, 16)

    def body(b, _):
        base = pl.multiple_of(wid * LPW + b * LB, 8)
        pltpu.sync_copy(sl_hbm.at[pl.ds(base, LB)], sidx)
        pltpu.sync_copy(dl_hbm.at[pl.ds(base, LB)], didx)
        cp1 = pltpu.async_copy(z_hbm.at[sidx], srows, sem)
        cp2 = pltpu.async_copy(z_hbm.at[didx], drows, sem)
        cp1.wait()
        cp2.wait()

        def group(g, _):
            row = g * 16 + iota
            acc = jnp.zeros((16,), jnp.float32)
            for c in range(D_OUT):
                col = jnp.full((16,), c, jnp.int32)
                acc = acc + (plsc.load_gather(srows, [row, col]) *
                             plsc.load_gather(drows, [row, col]))
            outv[pl.ds(g * 16, 16)] = acc
            return 0

        lax.fori_loop(0, LB // 16, group, 0)
        pltpu.sync_copy(outv, out_hbm.at[pl.ds(base, LB)])
        return 0

    lax.fori_loop(0, LBATCHES, body, 0)


def _decode(z, sl, dl):
    f = pl.kernel(
        _decode_body,
        out_type=jax.ShapeDtypeStruct((LPAD,), jnp.float32),
        mesh=_MESH,
        compiler_params=_SC_PARAMS_NL,
        scratch_types=[
            pltpu.VMEM((LB,), jnp.int32),
            pltpu.VMEM((LB,), jnp.int32),
            pltpu.VMEM((LB, D_OUT), jnp.float32),
            pltpu.VMEM((LB, D_OUT), jnp.float32),
            pltpu.VMEM((LB,), jnp.float32),
            pltpu.SemaphoreType.DMA,
        ],
    )
    return f(z, sl, dl)


_BLK = 1000


def _t1_body(d0_ref, d1_ref, emb_ref, w1_ref, hs_ref, dinv_ref):
    deg = d0_ref[:, 0:1] + d1_ref[:, 0:1] + 1.0
    dinv = lax.rsqrt(deg)
    h = jnp.dot(emb_ref[...], w1_ref[...], preferred_element_type=jnp.float32)
    hs_ref[...] = h * dinv
    dinv_ref[...] = jnp.broadcast_to(dinv, (_BLK, 16))


def _t1(d0, d1, emb, w1):
    return pl.pallas_call(
        _t1_body,
        grid=(N // _BLK,),
        in_specs=[
            pl.BlockSpec((_BLK, 16), lambda i: (i, 0)),
            pl.BlockSpec((_BLK, 16), lambda i: (i, 0)),
            pl.BlockSpec((_BLK, D_EMB), lambda i: (i, 0)),
            pl.BlockSpec((D_EMB, D_HID), lambda i: (0, 0)),
        ],
        out_specs=[
            pl.BlockSpec((_BLK, D_HID), lambda i: (i, 0)),
            pl.BlockSpec((_BLK, 16), lambda i: (i, 0)),
        ],
        out_shape=[
            jax.ShapeDtypeStruct((N, D_HID), jnp.float32),
            jax.ShapeDtypeStruct((N, 16), jnp.float32),
        ],
    )(d0, d1, emb, w1)


def _t2_body(p0_ref, p1_ref, hs1_ref, dinv_ref, b1_ref, w2_ref, hs2_ref):
    dinv = dinv_ref[:, 0:1]
    agg = (p0_ref[...] + p1_ref[...] + hs1_ref[...]) * dinv
    x2 = jnp.maximum(agg + b1_ref[...], 0.0)
    h2 = jnp.dot(x2, w2_ref[...], preferred_element_type=jnp.float32)
    hs2_ref[...] = h2 * dinv


def _t2(p0, p1, hs1, dinv, b1, w2):
    return pl.pallas_call(
        _t2_body,
        grid=(N // _BLK,),
        in_specs=[
            pl.BlockSpec((_BLK, D_HID), lambda i: (i, 0)),
            pl.BlockSpec((_BLK, D_HID), lambda i: (i, 0)),
            pl.BlockSpec((_BLK, D_HID), lambda i: (i, 0)),
            pl.BlockSpec((_BLK, 16), lambda i: (i, 0)),
            pl.BlockSpec((1, D_HID), lambda i: (0, 0)),
            pl.BlockSpec((D_HID, D_OUT), lambda i: (0, 0)),
        ],
        out_specs=pl.BlockSpec((_BLK, D_OUT), lambda i: (i, 0)),
        out_shape=jax.ShapeDtypeStruct((N, D_OUT), jnp.float32),
    )(p0, p1, hs1, dinv, b1, w2)


def _t3_body(q0_ref, q1_ref, hs2_ref, dinv_ref, b2_ref, z_ref):
    dinv = dinv_ref[:, 0:1]
    z_ref[...] = (q0_ref[...] + q1_ref[...] + hs2_ref[...]) * dinv + b2_ref[...]


def _t3(q0, q1, hs2, dinv, b2):
    return pl.pallas_call(
        _t3_body,
        grid=(N // _BLK,),
        in_specs=[
            pl.BlockSpec((_BLK, D_OUT), lambda i: (i, 0)),
            pl.BlockSpec((_BLK, D_OUT), lambda i: (i, 0)),
            pl.BlockSpec((_BLK, D_OUT), lambda i: (i, 0)),
            pl.BlockSpec((_BLK, 16), lambda i: (i, 0)),
            pl.BlockSpec((1, D_OUT), lambda i: (0, 0)),
        ],
        out_specs=pl.BlockSpec((_BLK, D_OUT), lambda i: (i, 0)),
        out_shape=jax.ShapeDtypeStruct((N, D_OUT), jnp.float32),
    )(q0, q1, hs2, dinv, b2)


def kernel(edge_index, edge_label_index, embedding, W1, b1, W2, b2):
    src = edge_index[0].astype(jnp.int32)
    dst = edge_index[1].astype(jnp.int32)
    npad_l = LPAD - edge_label_index.shape[1]
    sl = jnp.concatenate(
        [edge_label_index[0].astype(jnp.int32), jnp.zeros((npad_l,), jnp.int32)])
    dl = jnp.concatenate(
        [edge_label_index[1].astype(jnp.int32), jnp.zeros((npad_l,), jnp.int32)])

    deg = _degrees(dst)
    hs1, dinv = _t1(deg[0, :N], deg[1, :N], embedding, W1)
    p = _aggregate(hs1, src, dst, D_HID)
    hs2 = _t2(p[0, :N], p[1, :N], hs1, dinv, b1[None, :], W2)
    q = _aggregate(hs2, src, dst, D_OUT)
    z = _t3(q[0, :N], q[1, :N], hs2, dinv, b2[None, :])
    scores = _decode(z, sl, dl)
    return scores[:edge_label_index.shape[1]]


# speedup vs baseline: 8.4602x; 1.1462x over previous
import jax
import jax.numpy as jnp
from jax import lax
from jax.experimental import pallas as pl
from jax.experimental.pallas import tpu as pltpu
from jax.experimental.pallas import tpu_sc as plsc

N = 10000
NPAD = 10240
E = 320000
D_EMB = 128
D_HID = 128
D_OUT = 64

NC = 2
NS = 16
NW = NC * NS
ROWS_PER_TILE = NPAD // NS

EB = 128
ENB = 80
ENBC = 40
EPAD = NW * ENB * EB

LB = 128
LNB = 50
LPAD = NW * LNB * LB
LPW = LNB * LB

_MESH = plsc.VectorSubcoreMesh(
    core_axis_name="c", subcore_axis_name="s", num_cores=NC, num_subcores=NS)
_SC_PARAMS = pltpu.CompilerParams(use_tc_tiling_on_sc=False)
_SC_PARAMS_NL = pltpu.CompilerParams(
    use_tc_tiling_on_sc=False, needs_layout_passes=False)


def _wid():
    return lax.axis_index("s") * NC + lax.axis_index("c")


def _deg_body(dst_hbm, ones_hbm, zeros_hbm, out_hbm, table, onesv, idxd2):
    cid = lax.axis_index("c")
    sid = lax.axis_index("s")
    wid = _wid()
    r0 = sid * ROWS_PER_TILE
    pltpu.sync_copy(zeros_hbm.at[pl.ds(r0, ROWS_PER_TILE)],
                    table.at[pl.ds(r0, ROWS_PER_TILE)])
    pltpu.sync_copy(ones_hbm, onesv)
    pltpu.sync_copy(dst_hbm.at[pl.ds(wid * ENB, ENB)], idxd2)
    plsc.subcore_barrier()

    def body(j, _):
        pltpu.sync_copy(onesv, table.at[idxd2.at[j]], add=True)
        return 0

    lax.fori_loop(0, ENB, body, 0)
    plsc.subcore_barrier()
    pltpu.sync_copy(table.at[pl.ds(r0, ROWS_PER_TILE)],
                    out_hbm.at[cid, pl.ds(r0, ROWS_PER_TILE)])


def _degrees(dst2):
    ones = jnp.ones((EB, 16), jnp.float32)
    zeros = jnp.zeros((NPAD, 16), jnp.float32)
    f = pl.kernel(
        _deg_body,
        out_type=jax.ShapeDtypeStruct((NC, NPAD, 16), jnp.float32),
        mesh=_MESH,
        compiler_params=_SC_PARAMS,
        scratch_types=[
            pltpu.VMEM_SHARED((NPAD, 16), jnp.float32),
            pltpu.VMEM((EB, 16), jnp.float32),
            pltpu.VMEM((ENB, EB), jnp.int32),
        ],
    )
    return f(dst2, ones, zeros)


def _agg_body(hs_hbm, src_hbm, dst_hbm, zeros_hbm, out_hbm,
              table, idxs2, idxd2, rows0, rows1, sem0, sem1):
    cid = lax.axis_index("c")
    sid = lax.axis_index("s")
    wid = _wid()
    r0 = sid * ROWS_PER_TILE
    pltpu.sync_copy(zeros_hbm.at[pl.ds(r0, ROWS_PER_TILE)],
                    table.at[pl.ds(r0, ROWS_PER_TILE)])
    plsc.subcore_barrier()

    for c in range(ENB // ENBC):
        pltpu.sync_copy(src_hbm.at[pl.ds(wid * ENB + c * ENBC, ENBC)], idxs2)
        pltpu.sync_copy(dst_hbm.at[pl.ds(wid * ENB + c * ENBC, ENBC)], idxd2)
        pltpu.async_copy(hs_hbm.at[idxs2.at[0]], rows0, sem0)
        pltpu.async_copy(hs_hbm.at[idxs2.at[1]], rows1, sem1)

        def body(t, _):
            j0 = 2 * t
            pltpu.make_async_copy(hs_hbm.at[idxs2.at[j0]], rows0, sem0).wait()
            pltpu.sync_copy(rows0, table.at[idxd2.at[j0]], add=True)

            @pl.when(j0 + 2 < ENBC)
            def _():
                pltpu.async_copy(hs_hbm.at[idxs2.at[j0 + 2]], rows0, sem0)

            pltpu.make_async_copy(hs_hbm.at[idxs2.at[j0 + 1]], rows1, sem1).wait()
            pltpu.sync_copy(rows1, table.at[idxd2.at[j0 + 1]], add=True)

            @pl.when(j0 + 3 < ENBC)
            def _():
                pltpu.async_copy(hs_hbm.at[idxs2.at[j0 + 3]], rows1, sem1)

            return 0

        lax.fori_loop(0, ENBC // 2, body, 0)
    plsc.subcore_barrier()
    pltpu.sync_copy(table.at[pl.ds(r0, ROWS_PER_TILE)],
                    out_hbm.at[cid, pl.ds(r0, ROWS_PER_TILE)])


def _aggregate(hs, src2, dst2, d):
    zeros = jnp.zeros((NPAD, d), jnp.float32)
    f = pl.kernel(
        _agg_body,
        out_type=jax.ShapeDtypeStruct((NC, NPAD, d), jnp.float32),
        mesh=_MESH,
        compiler_params=_SC_PARAMS,
        scratch_types=[
            pltpu.VMEM_SHARED((NPAD, d), jnp.float32),
            pltpu.VMEM((ENBC, EB), jnp.int32),
            pltpu.VMEM((ENBC, EB), jnp.int32),
            pltpu.VMEM((EB, d), jnp.float32),
            pltpu.VMEM((EB, d), jnp.float32),
            pltpu.SemaphoreType.DMA,
            pltpu.SemaphoreType.DMA,
        ],
    )
    return f(hs, src2, dst2, zeros)


def _decode_body(z_hbm, sl_hbm, dl_hbm, out_hbm, sidx2, didx2,
                 srows0, drows0, srows1, drows1, outv, sems0, semd0, sems1, semd1):
    wid = _wid()
    iota = lax.iota(jnp.int32, 16)

    pltpu.sync_copy(sl_hbm.at[pl.ds(wid * LNB, LNB)], sidx2)
    pltpu.sync_copy(dl_hbm.at[pl.ds(wid * LNB, LNB)], didx2)

    pltpu.async_copy(z_hbm.at[sidx2.at[0]], srows0, sems0)
    pltpu.async_copy(z_hbm.at[didx2.at[0]], drows0, semd0)
    pltpu.async_copy(z_hbm.at[sidx2.at[1]], srows1, sems1)
    pltpu.async_copy(z_hbm.at[didx2.at[1]], drows1, semd1)

    def compute(srows, drows):
        def group(g, _):
            row = g * 16 + iota
            acc = jnp.zeros((16,), jnp.float32)
            for c in range(D_OUT):
                col = jnp.full((16,), c, jnp.int32)
                acc = acc + (plsc.load_gather(srows, [row, col]) *
                             plsc.load_gather(drows, [row, col]))
            outv[pl.ds(g * 16, 16)] = acc
            return 0

        lax.fori_loop(0, LB // 16, group, 0)

    def body(t, _):
        j0 = 2 * t
        pltpu.make_async_copy(z_hbm.at[sidx2.at[j0]], srows0, sems0).wait()
        pltpu.make_async_copy(z_hbm.at[didx2.at[j0]], drows0, semd0).wait()
        compute(srows0, drows0)
        pltpu.sync_copy(outv, out_hbm.at[pl.ds(wid * LPW + j0 * LB, LB)])

        @pl.when(j0 + 2 < LNB)
        def _():
            pltpu.async_copy(z_hbm.at[sidx2.at[j0 + 2]], srows0, sems0)
            pltpu.async_copy(z_hbm.at[didx2.at[j0 + 2]], drows0, semd0)

        pltpu.make_async_copy(z_hbm.at[sidx2.at[j0 + 1]], srows1, sems1).wait()
        pltpu.make_async_copy(z_hbm.at[didx2.at[j0 + 1]], drows1, semd1).wait()
        compute(srows1, drows1)
        pltpu.sync_copy(outv, out_hbm.at[pl.ds(wid * LPW + (j0 + 1) * LB, LB)])

        @pl.when(j0 + 3 < LNB)
        def _():
            pltpu.async_copy(z_hbm.at[sidx2.at[j0 + 3]], srows1, sems1)
            pltpu.async_copy(z_hbm.at[didx2.at[j0 + 3]], drows1, semd1)

        return 0

    lax.fori_loop(0, LNB // 2, body, 0)


def _decode(z, sl2, dl2):
    f = pl.kernel(
        _decode_body,
        out_type=jax.ShapeDtypeStruct((LPAD,), jnp.float32),
        mesh=_MESH,
        compiler_params=_SC_PARAMS_NL,
        scratch_types=[
            pltpu.VMEM((LNB, LB), jnp.int32),
            pltpu.VMEM((LNB, LB), jnp.int32),
            pltpu.VMEM((LB, D_OUT), jnp.float32),
            pltpu.VMEM((LB, D_OUT), jnp.float32),
            pltpu.VMEM((LB, D_OUT), jnp.float32),
            pltpu.VMEM((LB, D_OUT), jnp.float32),
            pltpu.VMEM((LB,), jnp.float32),
            pltpu.SemaphoreType.DMA,
            pltpu.SemaphoreType.DMA,
            pltpu.SemaphoreType.DMA,
            pltpu.SemaphoreType.DMA,
        ],
    )
    return f(z, sl2, dl2)


_BLK = 1000


def _t1_body(d0_ref, d1_ref, emb_ref, w1_ref, hs_ref, dinv_ref):
    deg = d0_ref[:, 0:1] + d1_ref[:, 0:1] + 1.0
    dinv = lax.rsqrt(deg)
    h = jnp.dot(emb_ref[...], w1_ref[...], preferred_element_type=jnp.float32)
    hs_ref[...] = h * dinv
    dinv_ref[...] = jnp.broadcast_to(dinv, (_BLK, 16))


def _t1(d0, d1, emb, w1):
    return pl.pallas_call(
        _t1_body,
        grid=(N // _BLK,),
        in_specs=[
            pl.BlockSpec((_BLK, 16), lambda i: (i, 0)),
            pl.BlockSpec((_BLK, 16), lambda i: (i, 0)),
            pl.BlockSpec((_BLK, D_EMB), lambda i: (i, 0)),
            pl.BlockSpec((D_EMB, D_HID), lambda i: (0, 0)),
        ],
        out_specs=[
            pl.BlockSpec((_BLK, D_HID), lambda i: (i, 0)),
            pl.BlockSpec((_BLK, 16), lambda i: (i, 0)),
        ],
        out_shape=[
            jax.ShapeDtypeStruct((N, D_HID), jnp.float32),
            jax.ShapeDtypeStruct((N, 16), jnp.float32),
        ],
    )(d0, d1, emb, w1)


def _t2_body(p0_ref, p1_ref, hs1_ref, dinv_ref, b1_ref, w2_ref, hs2_ref):
    dinv = dinv_ref[:, 0:1]
    agg = (p0_ref[...] + p1_ref[...] + hs1_ref[...]) * dinv
    x2 = jnp.maximum(agg + b1_ref[...], 0.0)
    h2 = jnp.dot(x2, w2_ref[...], preferred_element_type=jnp.float32)
    hs2_ref[...] = h2 * dinv


def _t2(p0, p1, hs1, dinv, b1, w2):
    return pl.pallas_call(
        _t2_body,
        grid=(N // _BLK,),
        in_specs=[
            pl.BlockSpec((_BLK, D_HID), lambda i: (i, 0)),
            pl.BlockSpec((_BLK, D_HID), lambda i: (i, 0)),
            pl.BlockSpec((_BLK, D_HID), lambda i: (i, 0)),
            pl.BlockSpec((_BLK, 16), lambda i: (i, 0)),
            pl.BlockSpec((1, D_HID), lambda i: (0, 0)),
            pl.BlockSpec((D_HID, D_OUT), lambda i: (0, 0)),
        ],
        out_specs=pl.BlockSpec((_BLK, D_OUT), lambda i: (i, 0)),
        out_shape=jax.ShapeDtypeStruct((N, D_OUT), jnp.float32),
    )(p0, p1, hs1, dinv, b1, w2)


def _t3_body(q0_ref, q1_ref, hs2_ref, dinv_ref, b2_ref, z_ref):
    dinv = dinv_ref[:, 0:1]
    z_ref[...] = (q0_ref[...] + q1_ref[...] + hs2_ref[...]) * dinv + b2_ref[...]


def _t3(q0, q1, hs2, dinv, b2):
    return pl.pallas_call(
        _t3_body,
        grid=(N // _BLK,),
        in_specs=[
            pl.BlockSpec((_BLK, D_OUT), lambda i: (i, 0)),
            pl.BlockSpec((_BLK, D_OUT), lambda i: (i, 0)),
            pl.BlockSpec((_BLK, D_OUT), lambda i: (i, 0)),
            pl.BlockSpec((_BLK, 16), lambda i: (i, 0)),
            pl.BlockSpec((1, D_OUT), lambda i: (0, 0)),
        ],
        out_specs=pl.BlockSpec((_BLK, D_OUT), lambda i: (i, 0)),
        out_shape=jax.ShapeDtypeStruct((N, D_OUT), jnp.float32),
    )(q0, q1, hs2, dinv, b2)


def kernel(edge_index, edge_label_index, embedding, W1, b1, W2, b2):
    epad = EPAD - E
    src2 = jnp.concatenate(
        [edge_index[0].astype(jnp.int32), jnp.zeros((epad,), jnp.int32)]
    ).reshape(NW * ENB, EB)
    dst2 = jnp.concatenate(
        [edge_index[1].astype(jnp.int32), jnp.full((epad,), N, jnp.int32)]
    ).reshape(NW * ENB, EB)

    nl = edge_label_index.shape[1]
    lpad = LPAD - nl
    sl2 = jnp.concatenate(
        [edge_label_index[0].astype(jnp.int32), jnp.zeros((lpad,), jnp.int32)]
    ).reshape(NW * LNB, LB)
    dl2 = jnp.concatenate(
        [edge_label_index[1].astype(jnp.int32), jnp.zeros((lpad,), jnp.int32)]
    ).reshape(NW * LNB, LB)

    deg = _degrees(dst2)
    hs1, dinv = _t1(deg[0, :N], deg[1, :N], embedding, W1)
    p = _aggregate(hs1, src2, dst2, D_HID)
    hs2 = _t2(p[0, :N], p[1, :N], hs1, dinv, b1[None, :], W2)
    q = _aggregate(hs2, src2, dst2, D_OUT)
    z = _t3(q[0, :N], q[1, :N], hs2, dinv, b2[None, :])
    scores = _decode(z, sl2, dl2)
    return scores[:nl]
